# baseline (speedup 1.0000x reference)
import functools

import jax
import jax.numpy as jnp
from jax import lax
from jax.experimental import pallas as pl
from jax.experimental.pallas import tpu as pltpu
from jax.experimental.pallas import tpu_sc as plsc

POOL = 64
LEN = 16
DIM = 2048
K = 2
BATCH = 1024

SUB = 8
ROW = LEN * DIM
SROW = ROW // SUB
POSITIONS = BATCH * K
NC, NS = 2, 16
NW = NC * NS
BPW = POSITIONS // NW
BT = 256


def _route_body(q_ref, k_ref, attn_ref, idx8_ref):
    q = q_ref[...]
    k = k_ref[...]
    sims = lax.dot_general(q, k, (((1,), (1,)), ((), ())),
                           preferred_element_type=jnp.float32)
    m1 = jnp.max(sims, axis=-1, keepdims=True)
    e = jnp.exp(sims - m1)
    attn_ref[...] = e / jnp.sum(e, axis=-1, keepdims=True)
    col = lax.broadcasted_iota(jnp.int32, sims.shape, 1)
    i1 = jnp.min(jnp.where(sims == m1, col, POOL), axis=-1, keepdims=True)
    sims2 = jnp.where(col == i1, -jnp.inf, sims)
    m2 = jnp.max(sims2, axis=-1, keepdims=True)
    i2 = jnp.min(jnp.where(sims2 == m2, col, POOL), axis=-1, keepdims=True)
    col16 = lax.broadcasted_iota(jnp.int32, (q.shape[0], K * SUB), 1)
    sel = jnp.where(col16 < SUB, i1, i2)
    idx8_ref[...] = sel * SUB + col16 % SUB


_route = pl.pallas_call(
    _route_body,
    grid=(BATCH // BT,),
    in_specs=[
        pl.BlockSpec((BT, DIM), lambda i: (i, 0)),
        pl.BlockSpec((POOL, DIM), lambda i: (0, 0)),
    ],
    out_specs=[
        pl.BlockSpec((BT, POOL), lambda i: (i, 0)),
        pl.BlockSpec((BT, K * SUB), lambda i: (i, 0)),
    ],
    out_shape=[
        jax.ShapeDtypeStruct((BATCH, POOL), jnp.float32),
        jax.ShapeDtypeStruct((BATCH, K * SUB), jnp.int32),
    ],
)


def _sc_gather_body(table, fidx, out, idx_v, buf0, buf1, g0, g1, p0, p1):
    wid = lax.axis_index("s") * NC + lax.axis_index("c")
    base = wid * BPW
    pltpu.sync_copy(fidx.at[pl.ds(base * SUB, BPW * SUB)], idx_v)
    bufs = (buf0, buf1)
    gs = (g0, g1)
    ps = (p0, p1)
    for b in range(2):
        pltpu.async_copy(table.at[idx_v.at[pl.ds(b * SUB, SUB)]],
                         bufs[b], gs[b])

    def body(j, carry):
        for b in range(2):
            i = 2 * j + b
            pltpu.make_async_copy(table.at[idx_v.at[pl.ds(i * SUB, SUB)]],
                                  bufs[b], gs[b]).wait()
            pltpu.async_copy(bufs[b], out.at[pl.ds((base + i) * SUB, SUB)],
                             ps[b]).wait()

            @pl.when(i + 2 < BPW)
            def _():
                pltpu.async_copy(
                    table.at[idx_v.at[pl.ds((i + 2) * SUB, SUB)]],
                    bufs[b], gs[b])
        return carry

    lax.fori_loop(0, BPW // 2, body, 0)


@functools.cache
def _make_sc_gather():
    return pl.kernel(
        _sc_gather_body,
        out_type=jax.ShapeDtypeStruct((POSITIONS * SUB, SROW), jnp.float32),
        mesh=plsc.VectorSubcoreMesh(core_axis_name="c", subcore_axis_name="s",
                                    num_cores=NC, num_subcores=NS),
        scratch_types=[
            pltpu.VMEM((BPW * SUB,), jnp.int32),
            pltpu.VMEM((SUB, SROW), jnp.float32),
            pltpu.VMEM((SUB, SROW), jnp.float32),
            pltpu.SemaphoreType.DMA,
            pltpu.SemaphoreType.DMA,
            pltpu.SemaphoreType.DMA,
            pltpu.SemaphoreType.DMA,
        ],
    )


@jax.jit
def kernel(query, prompts, keys):
    attn, idx8 = _route(query, keys)
    table = prompts.reshape(POOL * SUB, SROW)
    flat_idx = idx8.reshape(POSITIONS * SUB)
    out = _make_sc_gather()(table, flat_idx)
    selected = out.reshape(BATCH, K * LEN, DIM)
    return selected, attn
